# Initial kernel scaffold; baseline (speedup 1.0000x reference)
import jax
import jax.numpy as jnp
from jax.experimental import pallas as pl


def kernel(x, pos, batch, params):
    raise NotImplementedError("write your pallas kernel here")


# speedup vs baseline: 2.2457x; 2.2457x over previous
import functools

import jax
import jax.numpy as jnp
import numpy as np
from jax import lax
from jax.experimental import pallas as pl
from jax.experimental.pallas import tpu as pltpu

K_NN = 10
NGRAPH = 64
BN = 256
BC = 256
NBW = 16
_F32 = jnp.float32
_INF = np.float32(np.inf)
_IMAX = np.int32(np.iinfo(np.int32).max)


def _leaky(h):
    return jnp.where(h >= 0, h, 0.1 * h)


def _nt_dot(a, b):
    return lax.dot_general(a, b, (((1,), (1,)), ((), ())),
                           preferred_element_type=_F32)


def _nn_dot(a, b):
    return lax.dot_general(a, b, (((1,), (0,)), ((), ())),
                           preferred_element_type=_F32)


def _nn_dot_exact(a, b):
    return lax.dot_general(a, b, (((1,), (0,)), ((), ())),
                           preferred_element_type=_F32,
                           precision=lax.Precision.HIGHEST)


def _knn_body(cs_ref, ce_ref, feat_ref, x2c_ref, x23_ref, lo_ref, hi_ref,
              out_ref):
    blk = pl.program_id(0)
    q0 = blk * BN
    q = feat_ref[pl.ds(q0, BN), :]
    q2 = x2c_ref[pl.ds(q0, BN), :]
    lo = lo_ref[pl.ds(q0, BN), :]
    hi = hi_ref[pl.ds(q0, BN), :]
    rowid = q0 + lax.broadcasted_iota(jnp.int32, (BN, 1), 0)
    cs = cs_ref[blk]
    c0 = (cs // BC) * BC
    nch = (ce_ref[blk] - c0 + BC - 1) // BC

    def chunk(t, carry):
        best_d, best_i = carry
        cw = c0 // BC + t
        cb = cw * BC
        cand = feat_ref[pl.ds(cb, BC), :]
        c2 = x23_ref[cw]
        mm = _nt_dot(q, cand)
        d2 = (q2 + c2) - 2.0 * mm
        colid = cb + lax.broadcasted_iota(jnp.int32, (BN, BC), 1)
        bad = (colid < lo) | (colid >= hi) | (colid == rowid)
        d2 = jnp.where(bad, _INF, d2)
        d_all = jnp.concatenate([best_d, d2], axis=1)
        i_all = jnp.concatenate([best_i, colid], axis=1)
        bd, bi = [], []
        for _ in range(K_NN):
            m = jnp.min(d_all, axis=1, keepdims=True)
            sel = jnp.where(d_all == m, i_all, _IMAX)
            j = jnp.min(sel, axis=1, keepdims=True)
            bd.append(m)
            bi.append(j)
            d_all = jnp.where(i_all == j, _INF, d_all)
        pad_d = jnp.full((BN, NBW - K_NN), _INF, _F32)
        pad_i = jnp.full((BN, NBW - K_NN), _IMAX, jnp.int32)
        return (jnp.concatenate(bd + [pad_d], axis=1),
                jnp.concatenate(bi + [pad_i], axis=1))

    init = (jnp.full((BN, NBW), _INF, _F32),
            jnp.full((BN, NBW), _IMAX, jnp.int32))
    _, best_i = lax.fori_loop(0, nch, chunk, init)
    out_ref[...] = best_i


def _knn(feat, lo2, hi2, csb, ceb, nblk, np_):
    d = feat.shape[1]
    x2 = jnp.sum(feat * feat, axis=1)
    x2c = x2.reshape(np_, 1)
    x23 = x2.reshape(np_ // BC, 1, BC)

    def full(shape):
        return pl.BlockSpec(shape, lambda b, *_: tuple(0 for _ in shape))

    grid_spec = pltpu.PrefetchScalarGridSpec(
        num_scalar_prefetch=2,
        grid=(nblk,),
        in_specs=[full((np_, d)), full((np_, 1)), full((np_ // BC, 1, BC)),
                  full((np_, 1)), full((np_, 1))],
        out_specs=pl.BlockSpec((BN, NBW), lambda b, *_: (b, 0)),
    )
    return pl.pallas_call(
        _knn_body, grid_spec=grid_spec,
        out_shape=jax.ShapeDtypeStruct((np_, NBW), jnp.int32),
    )(csb, ceb, feat, x2c, x23, lo2, hi2)


def _gather_z1_body(f, cs_ref, ce_ref, x_ref, nb_ref, w1_ref, b1_ref,
                    z_ref, xj_scr):
    blk = pl.program_id(0)
    row0 = blk * BN
    xb = x_ref[pl.ds(row0, BN), :]
    idx = nb_ref[...]
    xj_scr[...] = jnp.zeros_like(xj_scr)
    cs = cs_ref[blk]
    c0 = (cs // BC) * BC
    nch = (ce_ref[blk] - c0 + BC - 1) // BC

    def chunk(t, carry):
        cb = c0 + t * BC
        win = x_ref[pl.ds(cb, BC), :]
        colmat = cb + lax.broadcasted_iota(jnp.int32, (BN, BC), 1)
        for j in range(K_NN):
            oh = (idx[:, j:j + 1] == colmat).astype(_F32)
            xj_scr[:, j * f:(j + 1) * f] += _nn_dot_exact(oh, win)
        return carry

    lax.fori_loop(0, nch, chunk, 0)
    w1 = w1_ref[...]
    b1 = b1_ref[...]
    for j in range(K_NN):
        xj = xj_scr[:, j * f:(j + 1) * f]
        e = jnp.concatenate([xb, xj - xb], axis=1)
        z_ref[:, j, :] = _nn_dot(e, w1) + b1


def _mid_body(z_ref, m_ref, v_ref, g_ref, be_ref, w2_ref, b2_ref, eps_ref,
              o_ref):
    sq = jnp.sqrt(v_ref[...] + eps_ref[0, 0])
    w2 = w2_ref[...]
    for j in range(K_NN):
        z = z_ref[:, j, :]
        a = _leaky((z - m_ref[...]) / sq * g_ref[...] + be_ref[...])
        o_ref[:, j, :] = _nn_dot(a, w2) + b2_ref[...]


def _out_body(final, nblk, z_ref, m_ref, v_ref, g_ref, be_ref, w3_ref,
              b3_ref, eps_ref, *rest):
    blk = pl.program_id(0)
    o = w3_ref.shape[1]
    eps = eps_ref[0, 0]
    sq = jnp.sqrt(v_ref[...] + eps)
    w3 = w3_ref[...]
    mx = jnp.full((BN, o), -_INF, _F32)
    for j in range(K_NN):
        a = _leaky((z_ref[:, j, :] - m_ref[...]) / sq * g_ref[...]
                   + be_ref[...])
        mx = jnp.maximum(mx, _nn_dot(a, w3) + b3_ref[...])
    if not final:
        out_ref, = rest
        out_ref[...] = mx
    else:
        (batch_ref, fw1_ref, fw2_ref, fw3_ref, fvecs_ref, fb3_ref,
         out_ref, gp_ref) = rest

        @pl.when(blk == 0)
        def _():
            gp_ref[...] = jnp.zeros_like(gp_ref)

        bt = batch_ref[0]
        ohT = (lax.broadcasted_iota(jnp.int32, (NGRAPH, 1), 0)
               == bt).astype(_F32)
        gp_ref[...] += _nn_dot_exact(ohT, mx)

        @pl.when(blk == nblk - 1)
        def _():
            def bn_cols(z, gv, bv):
                m = jnp.mean(z, axis=0, keepdims=True)
                v = jnp.mean((z - m) * (z - m), axis=0, keepdims=True)
                return (z - m) / jnp.sqrt(v + eps) * gv + bv

            g = gp_ref[...]
            z1 = _nn_dot(g, fw1_ref[...]) + fvecs_ref[0:1, :]
            a1 = _leaky(bn_cols(z1, fvecs_ref[1:2, :], fvecs_ref[2:3, :]))
            z2 = _nn_dot(a1, fw2_ref[...]) + fvecs_ref[3:4, :]
            a2 = _leaky(bn_cols(z2, fvecs_ref[4:5, :], fvecs_ref[5:6, :]))
            out_ref[...] = _nn_dot(a2, fw3_ref[...]) + fb3_ref[...]


def _edge_conv(pp, x, nb, csb, ceb, n_real, nblk, np_,
               batch3d=None, pfin=None, eps=1e-5):
    f = x.shape[1]
    h = pp['w1'].shape[1]
    o = pp['w3'].shape[1]
    n_edge = n_real * K_NN
    final = pfin is not None
    epsv = jnp.full((1, 1), eps, _F32)

    def fullspec(shape, extra=0):
        return pl.BlockSpec(shape, lambda b, *_: tuple(0 for _ in shape))

    gs1 = pltpu.PrefetchScalarGridSpec(
        num_scalar_prefetch=2, grid=(nblk,),
        in_specs=[fullspec((np_, f)),
                  pl.BlockSpec((BN, NBW), lambda b, *_: (b, 0)),
                  fullspec((2 * f, h)), fullspec((1, h))],
        out_specs=pl.BlockSpec((BN, K_NN, h), lambda b, *_: (b, 0, 0)),
        scratch_shapes=[pltpu.VMEM((BN, K_NN * f), _F32)],
    )
    z1 = pl.pallas_call(
        functools.partial(_gather_z1_body, f), grid_spec=gs1,
        out_shape=jax.ShapeDtypeStruct((np_, K_NN, h), _F32),
    )(csb, ceb, x, nb, pp['w1'], pp['b1'].reshape(1, h))

    z1e = lax.optimization_barrier(z1.reshape(np_ * K_NN, h)[:n_edge])
    m1 = jnp.mean(z1e, axis=0).reshape(1, h)
    v1 = jnp.var(z1e, axis=0).reshape(1, h)

    z2 = pl.pallas_call(
        _mid_body,
        grid=(nblk,),
        in_specs=[pl.BlockSpec((BN, K_NN, h), lambda b: (b, 0, 0)),
                  fullspec((1, h)), fullspec((1, h)), fullspec((1, h)),
                  fullspec((1, h)), fullspec((h, h)), fullspec((1, h)),
                  fullspec((1, 1))],
        out_specs=pl.BlockSpec((BN, K_NN, h), lambda b: (b, 0, 0)),
        out_shape=jax.ShapeDtypeStruct((np_, K_NN, h), _F32),
    )(z1, m1, v1, pp['g1'].reshape(1, h), pp['be1'].reshape(1, h),
      pp['w2'], pp['b2'].reshape(1, h), epsv)

    z2e = lax.optimization_barrier(z2.reshape(np_ * K_NN, h)[:n_edge])
    m2 = jnp.mean(z2e, axis=0).reshape(1, h)
    v2 = jnp.var(z2e, axis=0).reshape(1, h)

    ins = [z2, m2, v2, pp['g2'].reshape(1, h), pp['be2'].reshape(1, h),
           pp['w3'], pp['b3'].reshape(1, o), epsv]
    in_specs = [pl.BlockSpec((BN, K_NN, h), lambda b: (b, 0, 0)),
                fullspec((1, h)), fullspec((1, h)), fullspec((1, h)),
                fullspec((1, h)), fullspec((h, o)), fullspec((1, o)),
                fullspec((1, 1))]
    scratch = []
    if final:
        fh = pfin['w1'].shape[1]
        fvecs = jnp.stack([pfin['b1'], pfin['g1'], pfin['be1'],
                           pfin['b2'], pfin['g2'], pfin['be2']], axis=0)
        ins += [batch3d, pfin['w1'], pfin['w2'], pfin['w3'], fvecs,
                pfin['b3'].reshape(1, -1)]
        in_specs += [pl.BlockSpec((1, 1, BN), lambda b: (b, 0, 0)),
                     fullspec((o, fh)), fullspec((fh, fh)),
                     fullspec((fh, 1)), fullspec((6, fh)), fullspec((1, 1))]
        out_shape = jax.ShapeDtypeStruct((NGRAPH, 1), _F32)
        out_spec = pl.BlockSpec((NGRAPH, 1), lambda b: (0, 0))
        scratch = [pltpu.VMEM((NGRAPH, o), _F32)]
    else:
        out_shape = jax.ShapeDtypeStruct((np_, o), _F32)
        out_spec = pl.BlockSpec((BN, o), lambda b: (b, 0))
    return pl.pallas_call(
        functools.partial(_out_body, final, nblk),
        grid=(nblk,),
        in_specs=in_specs, out_specs=out_spec, out_shape=out_shape,
        scratch_shapes=scratch,
    )(*ins)


def kernel(x, pos, batch, params):
    n = x.shape[0]
    nblk = (n + BN - 1) // BN
    np_ = nblk * BN
    padn = np_ - n
    batch = batch.astype(jnp.int32)
    if padn:
        bpad = jnp.concatenate(
            [batch, jnp.full((padn,), NGRAPH, jnp.int32)])
        xp = jnp.pad(x, ((0, padn), (0, 0)))
        posp = jnp.pad(pos, ((0, padn), (0, 0)))
    else:
        bpad, xp, posp = batch, x, pos

    gids = jnp.arange(NGRAPH + 1, dtype=jnp.int32)
    seg_lo = jnp.searchsorted(bpad, gids, side='left').astype(jnp.int32)
    seg_hi = jnp.searchsorted(bpad, gids, side='right').astype(jnp.int32)
    row_lo = seg_lo[bpad]
    row_hi = seg_hi[bpad]
    csb = row_lo[0::BN]
    ceb = row_hi[BN - 1::BN]
    lo2 = row_lo.reshape(np_, 1)
    hi2 = row_hi.reshape(np_, 1)
    batch3d = bpad.reshape(nblk, 1, BN)

    nb1 = _knn(posp, lo2, hi2, csb, ceb, nblk, np_)
    h0 = jnp.concatenate([xp, posp], axis=1)
    h1 = _edge_conv(params['conv1'], h0, nb1, csb, ceb, n, nblk, np_)
    h2 = _edge_conv(params['conv2'], h1, nb1, csb, ceb, n, nblk, np_)
    nb2 = _knn(h2, lo2, hi2, csb, ceb, nblk, np_)
    out = _edge_conv(params['conv3'], h2, nb2, csb, ceb, n, nblk, np_,
                     batch3d=batch3d, pfin=params['out'])
    return out.reshape(NGRAPH)


# speedup vs baseline: 2.6062x; 1.1605x over previous
import functools

import jax
import jax.numpy as jnp
import numpy as np
from jax import lax
from jax.experimental import pallas as pl
from jax.experimental.pallas import tpu as pltpu

K_NN = 10
NGRAPH = 64
BN = 256
BC = 256
NBW = 16
_F32 = jnp.float32
_INF = np.float32(np.inf)
_IMAX = np.int32(np.iinfo(np.int32).max)


def _leaky(h):
    return jnp.where(h >= 0, h, 0.1 * h)


def _nt_dot(a, b):
    return lax.dot_general(a, b, (((1,), (1,)), ((), ())),
                           preferred_element_type=_F32)


def _nn_dot(a, b):
    return lax.dot_general(a, b, (((1,), (0,)), ((), ())),
                           preferred_element_type=_F32)


def _nn_dot_exact(a, b):
    return lax.dot_general(a, b, (((1,), (0,)), ((), ())),
                           preferred_element_type=_F32,
                           precision=lax.Precision.HIGHEST)


def _knn_body(cs_ref, ce_ref, feat_ref, x2c_ref, x23_ref, lo_ref, hi_ref,
              out_ref):
    blk = pl.program_id(0)
    q0 = blk * BN
    q = feat_ref[pl.ds(q0, BN), :]
    q2 = x2c_ref[pl.ds(q0, BN), :]
    lo = lo_ref[pl.ds(q0, BN), :]
    hi = hi_ref[pl.ds(q0, BN), :]
    rowid = q0 + lax.broadcasted_iota(jnp.int32, (BN, 1), 0)
    cs = cs_ref[blk]
    c0 = (cs // BC) * BC
    nch = (ce_ref[blk] - c0 + BC - 1) // BC

    def chunk(t, carry):
        best_d, best_i = carry
        cw = c0 // BC + t
        cb = cw * BC
        cand = feat_ref[pl.ds(cb, BC), :]
        c2 = x23_ref[cw]
        mm = _nt_dot(q, cand)
        d2 = (q2 + c2) - 2.0 * mm
        colid = cb + lax.broadcasted_iota(jnp.int32, (BN, BC), 1)
        bad = (colid < lo) | (colid >= hi) | (colid == rowid)
        d2 = jnp.where(bad, _INF, d2)
        d_all = jnp.concatenate([best_d, d2], axis=1)
        i_all = jnp.concatenate([best_i, colid], axis=1)
        bd, bi = [], []
        for _ in range(K_NN):
            m = jnp.min(d_all, axis=1, keepdims=True)
            sel = jnp.where(d_all == m, i_all, _IMAX)
            j = jnp.min(sel, axis=1, keepdims=True)
            bd.append(m)
            bi.append(j)
            d_all = jnp.where(i_all == j, _INF, d_all)
        pad_d = jnp.full((BN, NBW - K_NN), _INF, _F32)
        pad_i = jnp.full((BN, NBW - K_NN), _IMAX, jnp.int32)
        return (jnp.concatenate(bd + [pad_d], axis=1),
                jnp.concatenate(bi + [pad_i], axis=1))

    init = (jnp.full((BN, NBW), _INF, _F32),
            jnp.full((BN, NBW), _IMAX, jnp.int32))
    _, best_i = lax.fori_loop(0, nch, chunk, init)
    out_ref[...] = best_i


def _knn(feat, lo2, hi2, csb, ceb, nblk, np_):
    d = feat.shape[1]
    x2 = jnp.sum(feat * feat, axis=1)
    x2c = x2.reshape(np_, 1)
    x23 = x2.reshape(np_ // BC, 1, BC)

    def full(shape):
        return pl.BlockSpec(shape, lambda b, *_: tuple(0 for _ in shape))

    grid_spec = pltpu.PrefetchScalarGridSpec(
        num_scalar_prefetch=2,
        grid=(nblk,),
        in_specs=[full((np_, d)), full((np_, 1)), full((np_ // BC, 1, BC)),
                  full((np_, 1)), full((np_, 1))],
        out_specs=pl.BlockSpec((BN, NBW), lambda b, *_: (b, 0)),
    )
    return pl.pallas_call(
        _knn_body, grid_spec=grid_spec,
        out_shape=jax.ShapeDtypeStruct((np_, NBW), jnp.int32),
    )(csb, ceb, feat, x2c, x23, lo2, hi2)


def _gather_z1_body(f, cs_ref, ce_ref, x_ref, nb_ref, w1_ref, b1_ref,
                    z_ref, xj_scr):
    blk = pl.program_id(0)
    row0 = blk * BN
    xb = x_ref[pl.ds(row0, BN), :]
    idx = nb_ref[...]
    xj_scr[...] = jnp.zeros_like(xj_scr)
    cs = cs_ref[blk]
    c0 = (cs // BC) * BC
    nch = (ce_ref[blk] - c0 + BC - 1) // BC

    def chunk(t, carry):
        cb = c0 + t * BC
        win = x_ref[pl.ds(cb, BC), :]
        colmat = cb + lax.broadcasted_iota(jnp.int32, (BN, BC), 1)
        oh = jnp.concatenate(
            [(idx[:, j:j + 1] == colmat).astype(_F32) for j in range(K_NN)],
            axis=0)
        xj_scr[...] += _nn_dot_exact(oh, win)
        return carry

    lax.fori_loop(0, nch, chunk, 0)
    es = []
    for j in range(K_NN):
        xj = xj_scr[j * BN:(j + 1) * BN, :]
        es.append(jnp.concatenate([xb, xj - xb], axis=1))
    e_all = jnp.concatenate(es, axis=0)
    z_all = _nn_dot(e_all, w1_ref[...]) + b1_ref[...]
    for j in range(K_NN):
        z_ref[:, j, :] = z_all[j * BN:(j + 1) * BN, :]


def _mid_body(z_ref, m_ref, v_ref, g_ref, be_ref, w2_ref, b2_ref, eps_ref,
              o_ref):
    sq = jnp.sqrt(v_ref[...] + eps_ref[0, 0])
    z_all = jnp.concatenate([z_ref[:, j, :] for j in range(K_NN)], axis=0)
    a = _leaky((z_all - m_ref[...]) / sq * g_ref[...] + be_ref[...])
    o_all = _nn_dot(a, w2_ref[...]) + b2_ref[...]
    for j in range(K_NN):
        o_ref[:, j, :] = o_all[j * BN:(j + 1) * BN, :]


def _out_body(final, nblk, z_ref, m_ref, v_ref, g_ref, be_ref, w3_ref,
              b3_ref, eps_ref, *rest):
    blk = pl.program_id(0)
    o = w3_ref.shape[1]
    eps = eps_ref[0, 0]
    sq = jnp.sqrt(v_ref[...] + eps)
    z_all = jnp.concatenate([z_ref[:, j, :] for j in range(K_NN)], axis=0)
    a = _leaky((z_all - m_ref[...]) / sq * g_ref[...] + be_ref[...])
    o_all = _nn_dot(a, w3_ref[...]) + b3_ref[...]
    mx = o_all[0:BN, :]
    for j in range(1, K_NN):
        mx = jnp.maximum(mx, o_all[j * BN:(j + 1) * BN, :])
    if not final:
        out_ref, = rest
        out_ref[...] = mx
    else:
        (batch_ref, fw1_ref, fw2_ref, fw3_ref, fvecs_ref, fb3_ref,
         out_ref, gp_ref) = rest

        @pl.when(blk == 0)
        def _():
            gp_ref[...] = jnp.zeros_like(gp_ref)

        bt = batch_ref[0]
        ohT = (lax.broadcasted_iota(jnp.int32, (NGRAPH, 1), 0)
               == bt).astype(_F32)
        gp_ref[...] += _nn_dot_exact(ohT, mx)

        @pl.when(blk == nblk - 1)
        def _():
            def bn_cols(z, gv, bv):
                m = jnp.mean(z, axis=0, keepdims=True)
                v = jnp.mean((z - m) * (z - m), axis=0, keepdims=True)
                return (z - m) / jnp.sqrt(v + eps) * gv + bv

            g = gp_ref[...]
            z1 = _nn_dot(g, fw1_ref[...]) + fvecs_ref[0:1, :]
            a1 = _leaky(bn_cols(z1, fvecs_ref[1:2, :], fvecs_ref[2:3, :]))
            z2 = _nn_dot(a1, fw2_ref[...]) + fvecs_ref[3:4, :]
            a2 = _leaky(bn_cols(z2, fvecs_ref[4:5, :], fvecs_ref[5:6, :]))
            out_ref[...] = _nn_dot(a2, fw3_ref[...]) + fb3_ref[...]


def _edge_conv(pp, x, nb, csb, ceb, n_real, nblk, np_,
               batch3d=None, pfin=None, eps=1e-5):
    f = x.shape[1]
    h = pp['w1'].shape[1]
    o = pp['w3'].shape[1]
    n_edge = n_real * K_NN
    final = pfin is not None
    epsv = jnp.full((1, 1), eps, _F32)

    def fullspec(shape, extra=0):
        return pl.BlockSpec(shape, lambda b, *_: tuple(0 for _ in shape))

    gs1 = pltpu.PrefetchScalarGridSpec(
        num_scalar_prefetch=2, grid=(nblk,),
        in_specs=[fullspec((np_, f)),
                  pl.BlockSpec((BN, NBW), lambda b, *_: (b, 0)),
                  fullspec((2 * f, h)), fullspec((1, h))],
        out_specs=pl.BlockSpec((BN, K_NN, h), lambda b, *_: (b, 0, 0)),
        scratch_shapes=[pltpu.VMEM((K_NN * BN, f), _F32)],
    )
    z1 = pl.pallas_call(
        functools.partial(_gather_z1_body, f), grid_spec=gs1,
        out_shape=jax.ShapeDtypeStruct((np_, K_NN, h), _F32),
    )(csb, ceb, x, nb, pp['w1'], pp['b1'].reshape(1, h))

    z1e = lax.optimization_barrier(z1.reshape(np_ * K_NN, h)[:n_edge])
    m1 = jnp.mean(z1e, axis=0).reshape(1, h)
    v1 = jnp.var(z1e, axis=0).reshape(1, h)

    z2 = pl.pallas_call(
        _mid_body,
        grid=(nblk,),
        in_specs=[pl.BlockSpec((BN, K_NN, h), lambda b: (b, 0, 0)),
                  fullspec((1, h)), fullspec((1, h)), fullspec((1, h)),
                  fullspec((1, h)), fullspec((h, h)), fullspec((1, h)),
                  fullspec((1, 1))],
        out_specs=pl.BlockSpec((BN, K_NN, h), lambda b: (b, 0, 0)),
        out_shape=jax.ShapeDtypeStruct((np_, K_NN, h), _F32),
    )(z1, m1, v1, pp['g1'].reshape(1, h), pp['be1'].reshape(1, h),
      pp['w2'], pp['b2'].reshape(1, h), epsv)

    z2e = lax.optimization_barrier(z2.reshape(np_ * K_NN, h)[:n_edge])
    m2 = jnp.mean(z2e, axis=0).reshape(1, h)
    v2 = jnp.var(z2e, axis=0).reshape(1, h)

    ins = [z2, m2, v2, pp['g2'].reshape(1, h), pp['be2'].reshape(1, h),
           pp['w3'], pp['b3'].reshape(1, o), epsv]
    in_specs = [pl.BlockSpec((BN, K_NN, h), lambda b: (b, 0, 0)),
                fullspec((1, h)), fullspec((1, h)), fullspec((1, h)),
                fullspec((1, h)), fullspec((h, o)), fullspec((1, o)),
                fullspec((1, 1))]
    scratch = []
    if final:
        fh = pfin['w1'].shape[1]
        fvecs = jnp.stack([pfin['b1'], pfin['g1'], pfin['be1'],
                           pfin['b2'], pfin['g2'], pfin['be2']], axis=0)
        ins += [batch3d, pfin['w1'], pfin['w2'], pfin['w3'], fvecs,
                pfin['b3'].reshape(1, -1)]
        in_specs += [pl.BlockSpec((1, 1, BN), lambda b: (b, 0, 0)),
                     fullspec((o, fh)), fullspec((fh, fh)),
                     fullspec((fh, 1)), fullspec((6, fh)), fullspec((1, 1))]
        out_shape = jax.ShapeDtypeStruct((NGRAPH, 1), _F32)
        out_spec = pl.BlockSpec((NGRAPH, 1), lambda b: (0, 0))
        scratch = [pltpu.VMEM((NGRAPH, o), _F32)]
    else:
        out_shape = jax.ShapeDtypeStruct((np_, o), _F32)
        out_spec = pl.BlockSpec((BN, o), lambda b: (b, 0))
    return pl.pallas_call(
        functools.partial(_out_body, final, nblk),
        grid=(nblk,),
        in_specs=in_specs, out_specs=out_spec, out_shape=out_shape,
        scratch_shapes=scratch,
    )(*ins)


def kernel(x, pos, batch, params):
    n = x.shape[0]
    nblk = (n + BN - 1) // BN
    np_ = nblk * BN
    padn = np_ - n
    batch = batch.astype(jnp.int32)
    if padn:
        bpad = jnp.concatenate(
            [batch, jnp.full((padn,), NGRAPH, jnp.int32)])
        xp = jnp.pad(x, ((0, padn), (0, 0)))
        posp = jnp.pad(pos, ((0, padn), (0, 0)))
    else:
        bpad, xp, posp = batch, x, pos

    gids = jnp.arange(NGRAPH + 1, dtype=jnp.int32)
    seg_lo = jnp.searchsorted(bpad, gids, side='left').astype(jnp.int32)
    seg_hi = jnp.searchsorted(bpad, gids, side='right').astype(jnp.int32)
    row_lo = seg_lo[bpad]
    row_hi = seg_hi[bpad]
    csb = row_lo[0::BN]
    ceb = row_hi[BN - 1::BN]
    lo2 = row_lo.reshape(np_, 1)
    hi2 = row_hi.reshape(np_, 1)
    batch3d = bpad.reshape(nblk, 1, BN)

    nb1 = _knn(posp, lo2, hi2, csb, ceb, nblk, np_)
    h0 = jnp.concatenate([xp, posp], axis=1)
    h1 = _edge_conv(params['conv1'], h0, nb1, csb, ceb, n, nblk, np_)
    h2 = _edge_conv(params['conv2'], h1, nb1, csb, ceb, n, nblk, np_)
    nb2 = _knn(h2, lo2, hi2, csb, ceb, nblk, np_)
    out = _edge_conv(params['conv3'], h2, nb2, csb, ceb, n, nblk, np_,
                     batch3d=batch3d, pfin=params['out'])
    return out.reshape(NGRAPH)
